# baseline (speedup 1.0000x reference)
import jax
import jax.numpy as jnp
from jax import lax
from jax.experimental import pallas as pl
from jax.experimental.pallas import tpu as pltpu
from jax.experimental.pallas import tpu_sc as plsc

SEQ = 9
DM = 1024
EPS = 1e-7


def _renorm_body(table_hbm, enc_hbm, row_v, sq_v):
    wid = lax.axis_index("c") * 16 + lax.axis_index("s")

    @pl.when(wid < SEQ)
    def _():
        pltpu.sync_copy(table_hbm.at[wid], row_v)

        def sumsq(j, acc):
            v = row_v[pl.ds(j * 16, 16)]
            return acc + v * v

        acc = lax.fori_loop(0, DM // 16, sumsq, jnp.zeros((16,), jnp.float32))
        sq_v[...] = acc
        def lanesum(j, tot):
            return tot + plsc.load_gather(sq_v, [jnp.full((16,), j, jnp.int32)])

        nsq = lax.fori_loop(0, 16, lanesum, jnp.zeros((16,), jnp.float32))
        i = plsc.bitcast(nsq, jnp.int32)
        y = plsc.bitcast(jnp.int32(0x5F3759DF) - (i >> 1), jnp.float32)
        for _ in range(4):
            y = y * (1.5 - 0.5 * nsq * y * y)
        norm = nsq * y
        scale = jnp.where(norm > 1.0, 1.0 / (norm + EPS), jnp.float32(1.0))

        def scale_row(j, c):
            sl = pl.ds(j * 16, 16)
            row_v[sl] = row_v[sl] * scale
            return c

        lax.fori_loop(0, DM // 16, scale_row, 0)
        pltpu.sync_copy(row_v, enc_hbm.at[wid])


def _renorm_table_sc(table):
    return pl.kernel(
        _renorm_body,
        out_type=jax.ShapeDtypeStruct((SEQ, DM), jnp.float32),
        mesh=plsc.VectorSubcoreMesh(
            core_axis_name="c", subcore_axis_name="s", num_cores=1
        ),
        scratch_types=[
            pltpu.VMEM((DM,), jnp.float32),
            pltpu.VMEM((16,), jnp.float32),
        ],
        compiler_params=pltpu.CompilerParams(needs_layout_passes=False),
    )(table)


def _add_enc_kernel(x_ref, enc_ref, o_ref):
    sid = lax.rem(pl.program_id(0), SEQ)
    row = enc_ref[pl.ds(sid, 1), :]
    o_ref[...] = x_ref[...] + row[None]


def kernel(x, table):
    b, s, n, d = x.shape
    enc = _renorm_table_sc(table)
    xr = x.reshape(b * s, n, d)
    out = pl.pallas_call(
        _add_enc_kernel,
        grid=(b * s,),
        in_specs=[
            pl.BlockSpec((1, n, d), lambda i: (i, 0, 0)),
            pl.BlockSpec((SEQ, d), lambda i: (0, 0)),
        ],
        out_specs=pl.BlockSpec((1, n, d), lambda i: (i, 0, 0)),
        out_shape=jax.ShapeDtypeStruct((b * s, n, d), x.dtype),
        compiler_params=pltpu.CompilerParams(
            dimension_semantics=("arbitrary",),
            vmem_limit_bytes=60 * 1024 * 1024,
        ),
    )(xr, enc)
    return out.reshape(b, s, n, d)


# speedup vs baseline: 1.0023x; 1.0023x over previous
import jax
import jax.numpy as jnp
from jax import lax
from jax.experimental import pallas as pl
from jax.experimental.pallas import tpu as pltpu
from jax.experimental.pallas import tpu_sc as plsc

SEQ = 9
DM = 1024
EPS = 1e-7


def _renorm_body(table_hbm, enc_hbm, row_v, sq_v):
    wid = lax.axis_index("c") * 16 + lax.axis_index("s")

    @pl.when(wid < SEQ)
    def _():
        pltpu.sync_copy(table_hbm.at[wid], row_v)

        def sumsq(j, acc):
            v = row_v[pl.ds(j * 16, 16)]
            return acc + v * v

        acc = lax.fori_loop(0, DM // 16, sumsq, jnp.zeros((16,), jnp.float32))
        sq_v[...] = acc
        def lanesum(j, tot):
            return tot + plsc.load_gather(sq_v, [jnp.full((16,), j, jnp.int32)])

        nsq = lax.fori_loop(0, 16, lanesum, jnp.zeros((16,), jnp.float32))
        i = plsc.bitcast(nsq, jnp.int32)
        y = plsc.bitcast(jnp.int32(0x5F3759DF) - (i >> 1), jnp.float32)
        for _ in range(4):
            y = y * (1.5 - 0.5 * nsq * y * y)
        norm = nsq * y
        scale = jnp.where(norm > 1.0, 1.0 / (norm + EPS), jnp.float32(1.0))

        def scale_row(j, c):
            sl = pl.ds(j * 16, 16)
            row_v[sl] = row_v[sl] * scale
            return c

        lax.fori_loop(0, DM // 16, scale_row, 0)
        pltpu.sync_copy(row_v, enc_hbm.at[wid])


def _renorm_table_sc(table):
    return pl.kernel(
        _renorm_body,
        out_type=jax.ShapeDtypeStruct((SEQ, DM), jnp.float32),
        mesh=plsc.VectorSubcoreMesh(
            core_axis_name="c", subcore_axis_name="s", num_cores=1
        ),
        scratch_types=[
            pltpu.VMEM((DM,), jnp.float32),
            pltpu.VMEM((16,), jnp.float32),
        ],
        compiler_params=pltpu.CompilerParams(
            needs_layout_passes=False,
            skip_device_barrier=True,
        ),
    )(table)


def _add_enc_kernel(x_ref, enc_ref, o_ref):
    sid = lax.rem(pl.program_id(0), SEQ)
    row = enc_ref[pl.ds(sid, 1), :]
    o_ref[...] = x_ref[...] + row[None]


def kernel(x, table):
    b, s, n, d = x.shape
    enc = _renorm_table_sc(table)
    xr = x.reshape(b * s, n, d)
    out = pl.pallas_call(
        _add_enc_kernel,
        grid=(b * s,),
        in_specs=[
            pl.BlockSpec((1, n, d), lambda i: (i, 0, 0)),
            pl.BlockSpec((SEQ, d), lambda i: (0, 0)),
        ],
        out_specs=pl.BlockSpec((1, n, d), lambda i: (i, 0, 0)),
        out_shape=jax.ShapeDtypeStruct((b * s, n, d), x.dtype),
        compiler_params=pltpu.CompilerParams(
            dimension_semantics=("arbitrary",),
            vmem_limit_bytes=60 * 1024 * 1024,
        ),
    )(xr, enc)
    return out.reshape(b, s, n, d)
